# baseline (speedup 1.0000x reference)
import functools

import numpy as np
import jax
import jax.numpy as jnp
from jax import lax
from jax.experimental import pallas as pl
from jax.experimental.pallas import tpu as pltpu
from jax.experimental.pallas import tpu_sc as plsc

N = 10000
E = 320000
F = 128
H1 = 16
H2 = 8
C = 8
DIMD = 64
DMAX = 1000
OMEGA = 0.1
K_THRESH = 32

NC = 2
NS = 16
NW = NC * NS
EPW = E // NW
CH = 80
NCHUNK = EPW // CH

NP = 10240
NP8 = NP // 8
NCHUNK_N = NP // CH
CPT = NCHUNK_N // NS

G = 512
G8 = G // 8
G16 = G // 16
NG = NP // G


def _make_pe(d_max, dim):
    pos = np.arange(d_max)[:, None].astype(np.float32)
    div = np.exp(np.arange(0, dim, 2).astype(np.float32) * -(np.log(10000.0) / dim))
    pe = np.zeros((d_max, dim), dtype=np.float32)
    pe[:, 0::2] = np.sin(pos * div)
    pe[:, 1::2] = np.cos(pos * div)
    return pe

_PE = _make_pe(DMAX, DIMD)

_F32 = jnp.float32


def _zero_shared(zbuf, acc, sid):
    z16 = jnp.zeros((16,), _F32)

    @pl.loop(0, CH)
    def _(i):
        zbuf[i] = z16

    @pl.loop(0, CPT)
    def _(k):
        g = sid + k * NS
        pltpu.sync_copy(zbuf, acc.at[pl.ds(g * CH, CH)])


def _writeback(acc, out, sid, sec):
    @pl.loop(0, CPT)
    def _(k):
        g = sid + k * NS
        pltpu.sync_copy(acc.at[pl.ds(g * CH, CH)],
                        out.at[pl.ds(sec * NP + g * CH, CH)])


def _xform_vec(v):
    j = v & 511
    return (v - j) + ((j & 63) << 3) + (j >> 6)

NBUF = 4
NQ = (NCHUNK - 1) // NBUF


def _sc_agg_hist_body(msg_hbm, adj_hbm, idx_hbm, out_hbm,
                      adjb, tixb, rows, idxb, tix_i, ones_c, ones_i, zbuf,
                      acc, acc_cnt, accw, si, sg, ss, st):
    cid = lax.axis_index("c")
    sid = lax.axis_index("s")
    wid = cid * NS + sid
    base = wid * EPW

    e0 = jnp.where(lax.iota(jnp.int32, 16) == 0, 1.0, 0.0).astype(_F32)

    @pl.loop(0, CH)
    def _(i):
        ones_c[i] = e0

    _zero_shared(zbuf, acc, sid)
    _zero_shared(zbuf, acc_cnt, sid)

    @pl.when(cid == 0)
    def _():
        _zero_shared(zbuf, accw, sid)

    plsc.subcore_barrier()

    def idx_dma(c, s):
        return pltpu.make_async_copy(
            adj_hbm.at[:, pl.ds(base + c * CH, CH)], adjb.at[s], si.at[s])

    def xform(s):
        for part in range(2):
            for i in range(CH // 16):
                sl = pl.ds(i * 16, 16)
                tixb[s, part, sl] = _xform_vec(adjb[s, part, sl])

    def gat_dma(s):
        return pltpu.make_async_copy(
            msg_hbm.at[tixb.at[s, 0]], rows.at[s], sg.at[s])

    def scat_dma(s):
        return pltpu.make_async_copy(
            rows.at[s], acc.at[tixb.at[s, 1]], ss.at[s])

    def cnt_dma(s):
        return pltpu.make_async_copy(
            ones_c, acc_cnt.at[tixb.at[s, 1]], st.at[s])

    for s in range(NBUF):
        idx_dma(s, s).start()

    @pl.loop(0, NQ)
    def _(q):
        c0 = q * NBUF
        for s in range(NBUF):
            idx_dma(c0 + s, s).wait()
            xform(s)
            pltpu.async_copy(msg_hbm.at[tixb.at[s, 0]], rows.at[s], sg.at[s])
            pltpu.async_copy(ones_c, acc_cnt.at[tixb.at[s, 1]], st.at[s],
                             add=True)
        for s in range(NBUF):
            gat_dma(s).wait()
            pltpu.async_copy(rows.at[s], acc.at[tixb.at[s, 1]], ss.at[s],
                             add=True)
        for s in range(NBUF):
            scat_dma(s).wait()
            cnt_dma(s).wait()

            @pl.when(q < NQ - 1)
            def _():
                idx_dma(c0 + NBUF + s, s).start()

    idx_dma(NCHUNK - 1, 0).start()
    idx_dma(NCHUNK - 1, 0).wait()
    xform(0)
    pltpu.async_copy(msg_hbm.at[tixb.at[0, 0]], rows.at[0], sg.at[0])
    pltpu.async_copy(ones_c, acc_cnt.at[tixb.at[0, 1]], st.at[0], add=True)
    gat_dma(0).wait()
    pltpu.async_copy(rows.at[0], acc.at[tixb.at[0, 1]], ss.at[0], add=True)
    scat_dma(0).wait()
    cnt_dma(0).wait()

    @pl.when(cid == 0)
    def _():
        @pl.loop(0, 40)
        def _(i):
            ones_i[i] = e0

        for c in (sid, sid + NS):
            @pl.when(c < 25)
            def _():
                pltpu.sync_copy(idx_hbm.at[pl.ds(c * 40, 40)], idxb)
                for o in (0, 16, 24):
                    tix_i[pl.ds(o, 16)] = _xform_vec(idxb[pl.ds(o, 16)])
                pltpu.sync_copy(ones_i, accw.at[tix_i], add=True)

    plsc.subcore_barrier()

    @pl.when(cid == 0)
    def _():
        _writeback(acc, out_hbm, sid, 0)
        _writeback(acc_cnt, out_hbm, sid, 2)
        _writeback(accw, out_hbm, sid, 4)

    @pl.when(cid == 1)
    def _():
        _writeback(acc, out_hbm, sid, 1)
        _writeback(acc_cnt, out_hbm, sid, 3)


def _sc_agg_body(msg_hbm, adj_hbm, out_hbm,
                 adjb, tixb, rows, zbuf, acc, si, sg, ss):
    cid = lax.axis_index("c")
    sid = lax.axis_index("s")
    wid = cid * NS + sid
    base = wid * EPW

    _zero_shared(zbuf, acc, sid)
    plsc.subcore_barrier()

    def idx_dma(c, s):
        return pltpu.make_async_copy(
            adj_hbm.at[:, pl.ds(base + c * CH, CH)], adjb.at[s], si.at[s])

    def xform(s):
        for part in range(2):
            for i in range(CH // 16):
                sl = pl.ds(i * 16, 16)
                tixb[s, part, sl] = _xform_vec(adjb[s, part, sl])

    def gat_dma(s):
        return pltpu.make_async_copy(
            msg_hbm.at[tixb.at[s, 0]], rows.at[s], sg.at[s])

    def scat_dma(s):
        return pltpu.make_async_copy(
            rows.at[s], acc.at[tixb.at[s, 1]], ss.at[s])

    for s in range(NBUF):
        idx_dma(s, s).start()

    @pl.loop(0, NQ)
    def _(q):
        c0 = q * NBUF
        for s in range(NBUF):
            idx_dma(c0 + s, s).wait()
            xform(s)
            pltpu.async_copy(msg_hbm.at[tixb.at[s, 0]], rows.at[s], sg.at[s])
        for s in range(NBUF):
            gat_dma(s).wait()
            pltpu.async_copy(rows.at[s], acc.at[tixb.at[s, 1]], ss.at[s],
                             add=True)
        for s in range(NBUF):
            scat_dma(s).wait()

            @pl.when(q < NQ - 1)
            def _():
                idx_dma(c0 + NBUF + s, s).start()

    idx_dma(NCHUNK - 1, 0).start()
    idx_dma(NCHUNK - 1, 0).wait()
    xform(0)
    pltpu.async_copy(msg_hbm.at[tixb.at[0, 0]], rows.at[0], sg.at[0])
    gat_dma(0).wait()
    pltpu.async_copy(rows.at[0], acc.at[tixb.at[0, 1]], ss.at[0], add=True)
    scat_dma(0).wait()

    plsc.subcore_barrier()

    @pl.when(cid == 0)
    def _():
        _writeback(acc, out_hbm, sid, 0)

    @pl.when(cid == 1)
    def _():
        _writeback(acc, out_hbm, sid, 1)


@functools.lru_cache(maxsize=None)
def _sc_kernels():
    mesh = plsc.VectorSubcoreMesh(core_axis_name="c", subcore_axis_name="s")
    cp = pltpu.CompilerParams(use_tc_tiling_on_sc=False)
    agg_hist = pl.kernel(
        _sc_agg_hist_body,
        out_type=jax.ShapeDtypeStruct((5 * NP, 16), _F32),
        mesh=mesh,
        scratch_types=[pltpu.VMEM((NBUF, 2, CH), jnp.int32),
                       pltpu.VMEM((NBUF, 2, CH), jnp.int32),
                       pltpu.VMEM((NBUF, CH, 16), _F32),
                       pltpu.VMEM((40,), jnp.int32),
                       pltpu.VMEM((40,), jnp.int32),
                       pltpu.VMEM((CH, 16), _F32),
                       pltpu.VMEM((40, 16), _F32),
                       pltpu.VMEM((CH, 16), _F32),
                       pltpu.VMEM_SHARED((NP, 16), _F32),
                       pltpu.VMEM_SHARED((NP, 16), _F32),
                       pltpu.VMEM_SHARED((NP, 16), _F32),
                       pltpu.SemaphoreType.DMA((NBUF,)),
                       pltpu.SemaphoreType.DMA((NBUF,)),
                       pltpu.SemaphoreType.DMA((NBUF,)),
                       pltpu.SemaphoreType.DMA((NBUF,))],
        compiler_params=cp,
    )
    agg = pl.kernel(
        _sc_agg_body,
        out_type=jax.ShapeDtypeStruct((2 * NP, 16), _F32),
        mesh=mesh,
        scratch_types=[pltpu.VMEM((NBUF, 2, CH), jnp.int32),
                       pltpu.VMEM((NBUF, 2, CH), jnp.int32),
                       pltpu.VMEM((NBUF, CH, 16), _F32),
                       pltpu.VMEM((CH, 16), _F32),
                       pltpu.VMEM_SHARED((NP, 16), _F32),
                       pltpu.SemaphoreType.DMA((NBUF,)),
                       pltpu.SemaphoreType.DMA((NBUF,)),
                       pltpu.SemaphoreType.DMA((NBUF,))],
        compiler_params=cp,
    )
    return agg_hist, agg


def _sc_agg_hist(msg, adj, idx):
    return _sc_kernels()[0](msg, adj, idx)


def _sc_agg(msg, adj):
    return _sc_kernels()[1](msg, adj)


def _elu(v):
    return jnp.where(v > 0, v, jnp.exp(v) - 1.0)


def _groups():
    for g in range(NG):
        n0 = g * G
        yield g, n0, min(G, N - n0)


def _pad_rows(v, rows):
    n = v.shape[0]
    if n == rows:
        return v
    return jnp.concatenate([v, jnp.zeros((rows - n, v.shape[1]), v.dtype)], 0)


def _pack16(v):
    v = _pad_rows(v, G)
    return jnp.concatenate([v[64 * a:64 * a + 64, :] for a in range(8)], 1)


def _unpack16(raw, n):
    v = jnp.concatenate([raw[:, 16 * a:16 * a + 16] for a in range(8)], 0)
    return v[:n] if n < G else v


def _pack8(v):
    v = _pad_rows(v, G)
    return jnp.concatenate([v[32 * a:32 * a + 32, :] for a in range(16)], 1)


def _unpack8(raw, n):
    v = jnp.concatenate([raw[:, 8 * a:8 * a + 8] for a in range(16)], 0)
    return v[:n] if n < G else v


def _stage_a_body(x_ref, d_ref, wall_ref, pe_ref, wgb_ref, bgb_ref,
                  msg1_ref, h1_ref, gb2_ref, qa_ref):
    t = lax.dot_general(pe_ref[...], wgb_ref[...], (((1,), (0,)), ((), ())),
                        preferred_element_type=_F32)
    t = jnp.maximum(t + bgb_ref[...], 0.0)
    t_hi = t.astype(jnp.bfloat16)
    t_lo = (t - t_hi.astype(_F32)).astype(jnp.bfloat16)
    wall = wall_ref[...]

    for g, n0, n in _groups():
        xb = x_ref[pl.ds(n0, n), :]
        hxx = lax.dot_general(xb, wall, (((1,), (0,)), ((), ())),
                              preferred_element_type=_F32)
        h = hxx[:, :H1]
        xa = hxx[:, H1:2 * H1]
        xr = hxx[:, 2 * H1:3 * H1]

        db = d_ref[pl.ds(n0, n), :]
        oh = (db == lax.broadcasted_iota(jnp.int32, (n, DMAX), 1)
              ).astype(jnp.bfloat16)
        gb = (lax.dot_general(oh, t_hi, (((1,), (0,)), ((), ())),
                              preferred_element_type=_F32) +
              lax.dot_general(oh, t_lo, (((1,), (0,)), ((), ())),
                              preferred_element_type=_F32))
        g1 = gb[:, :H1]
        b1 = gb[:, H1:2 * H1]
        g2 = gb[:, 2 * H1:2 * H1 + H2]
        b2 = gb[:, 2 * H1 + H2:2 * H1 + 2 * H2]

        r = (db < K_THRESH).astype(_F32)
        badd = g1 * xa + b1
        brev = g1 * xr + b1
        ra = r * badd
        rr = (1.0 - r) * brev

        msg1_ref[pl.ds(g * G8, G8), :] = _pack16(h + OMEGA * (ra - rr))
        h1_ref[pl.ds(g * G8, G8), :] = _pack16(h)
        gb2_ref[pl.ds(g * G8, G8), :] = _pack16(
            jnp.concatenate([g2, b2], axis=1))

        qb1 = jnp.sum(ra * ra, axis=1, keepdims=True) + \
            jnp.sum(rr * rr, axis=1, keepdims=True)
        qf1 = jnp.sum(g1 * g1, axis=1, keepdims=True) + \
            jnp.sum(b1 * b1, axis=1, keepdims=True)
        qf2 = jnp.sum(g2 * g2, axis=1, keepdims=True) + \
            jnp.sum(b2 * b2, axis=1, keepdims=True)
        qa8 = jnp.concatenate(
            [qb1, qf1, qf2, r, jnp.zeros((n, 4), _F32)], axis=1)
        qa_ref[pl.ds(g * G16, G16), :] = _pack8(qa8)


def _stage_c_body(h1_ref, comb1_ref, gb2_ref, qa_ref, w2_ref,
                  msg2_ref, h2_ref, qb2_ref):
    w2 = w2_ref[...]
    for g, n0, n in _groups():
        h1pre = _unpack16(h1_ref[pl.ds(g * G8, G8), :], n)
        p0 = _unpack16(comb1_ref[pl.ds(0 * NP8 + g * G8, G8), :], n)
        p1 = _unpack16(comb1_ref[pl.ds(1 * NP8 + g * G8, G8), :], n)
        cnt = (_unpack16(comb1_ref[pl.ds(2 * NP8 + g * G8, G8), :], n)[:, 0:1] +
               _unpack16(comb1_ref[pl.ds(3 * NP8 + g * G8, G8), :], n)[:, 0:1])
        agg1 = (p0 + p1) / jnp.maximum(cnt, 1.0)
        h1 = _elu(jnp.concatenate([h1pre, agg1], axis=1))
        hxx = lax.dot_general(h1, w2, (((1,), (0,)), ((), ())),
                              preferred_element_type=_F32)
        h = hxx[:, :H2]
        xa = hxx[:, H2:2 * H2]
        xr = hxx[:, 2 * H2:3 * H2]

        gb2 = _unpack16(gb2_ref[pl.ds(g * G8, G8), :], n)
        g2 = gb2[:, :H2]
        b2 = gb2[:, H2:2 * H2]
        r = _unpack8(qa_ref[pl.ds(g * G16, G16), :], n)[:, 3:4]
        badd = g2 * xa + b2
        brev = g2 * xr + b2
        ra = r * badd
        rr = (1.0 - r) * brev

        msg2 = jnp.concatenate(
            [h + OMEGA * (ra - rr), jnp.zeros((n, 16 - H2), _F32)], axis=1)
        msg2_ref[pl.ds(g * G8, G8), :] = _pack16(msg2)
        h2_ref[pl.ds(g * G16, G16), :] = _pack8(h)
        qb2 = jnp.sum(ra * ra, axis=1, keepdims=True) + \
            jnp.sum(rr * rr, axis=1, keepdims=True)
        qb2_ref[pl.ds(g * G16, G16), :] = _pack8(
            jnp.concatenate([qb2, jnp.zeros((n, 7), _F32)], axis=1))


def _stage_d_body(h2_ref, comb2_ref, comb1_ref, qa_ref, qb2_ref,
                  wfc_ref, bfc_ref, logp_ref, bacc_ref, facc_ref):
    wfc = wfc_ref[...]
    bfc = bfc_ref[...]
    bsum = jnp.zeros((1, 1), _F32)
    fsum = jnp.zeros((1, 1), _F32)
    for g, n0, n in _groups():
        h2pre = _unpack8(h2_ref[pl.ds(g * G16, G16), :], n)
        p0 = _unpack16(comb2_ref[pl.ds(0 * NP8 + g * G8, G8), :], n)[:, :H2]
        p1 = _unpack16(comb2_ref[pl.ds(1 * NP8 + g * G8, G8), :], n)[:, :H2]
        cnt = (_unpack16(comb1_ref[pl.ds(2 * NP8 + g * G8, G8), :], n)[:, 0:1] +
               _unpack16(comb1_ref[pl.ds(3 * NP8 + g * G8, G8), :], n)[:, 0:1])
        agg2 = (p0 + p1) / jnp.maximum(cnt, 1.0)
        h2 = _elu(jnp.concatenate([h2pre, agg2], axis=1))
        logits = lax.dot_general(h2, wfc, (((1,), (0,)), ((), ())),
                                 preferred_element_type=_F32) + bfc
        m = jnp.max(logits, axis=1, keepdims=True)
        s = logits - m
        lse = jnp.log(jnp.sum(jnp.exp(s), axis=1, keepdims=True))
        logp_ref[pl.ds(n0, n), :] = s - lse

        wv = _unpack16(comb1_ref[pl.ds(4 * NP8 + g * G8, G8), :], n)[:, 0:1]
        qa8 = _unpack8(qa_ref[pl.ds(g * G16, G16), :], n)
        qb2 = _unpack8(qb2_ref[pl.ds(g * G16, G16), :], n)[:, 0:1]
        bsum += jnp.sum(wv * qa8[:, 0:1], keepdims=True) / (1000.0 * H1) + \
            jnp.sum(wv * qb2, keepdims=True) / (1000.0 * H2)
        fsum += jnp.sum(wv * qa8[:, 1:2], keepdims=True) / (1000.0 * H1) + \
            jnp.sum(wv * qa8[:, 2:3], keepdims=True) / (1000.0 * H2)
    bacc_ref[...] = bsum
    facc_ref[...] = fsum


def _stage_a(x, d2, wall, pe, wgb, bgb):
    return pl.pallas_call(
        _stage_a_body,
        out_shape=[jax.ShapeDtypeStruct((NP8, 128), _F32),
                   jax.ShapeDtypeStruct((NP8, 128), _F32),
                   jax.ShapeDtypeStruct((NP8, 128), _F32),
                   jax.ShapeDtypeStruct((NP8 // 2, 128), _F32)],
    )(x, d2, wall, pe, wgb, bgb)


def _stage_c(h1p, comb1p, gb2p, qap, w2cat):
    return pl.pallas_call(
        _stage_c_body,
        out_shape=[jax.ShapeDtypeStruct((NP8, 128), _F32),
                   jax.ShapeDtypeStruct((NP8 // 2, 128), _F32),
                   jax.ShapeDtypeStruct((NP8 // 2, 128), _F32)],
    )(h1p, comb1p, gb2p, qap, w2cat)


def _stage_d(h2p, comb2p, comb1p, qap, qb2p, wfc, bfc):
    return pl.pallas_call(
        _stage_d_body,
        out_shape=[jax.ShapeDtypeStruct((N, C), _F32),
                   jax.ShapeDtypeStruct((1, 1), _F32),
                   jax.ShapeDtypeStruct((1, 1), _F32)],
    )(h2p, comb2p, comb1p, qap, qb2p, wfc, bfc)


def kernel(x, adj, d, idx, edge, weight1, W_gamma1, W_beta1, b_gamma1,
           b_beta1, W_add1, W_rev1, weight2, W_gamma2, W_beta2, b_gamma2,
           b_beta2, W_add2, W_rev2, W_fc, b_fc):
    d2 = d.reshape(N, 1)
    pe = jnp.asarray(_PE)
    wall = jnp.concatenate([weight1, W_add1, W_rev1], axis=1)
    wgb = jnp.concatenate([W_gamma1, W_beta1, W_gamma2, W_beta2], axis=1)
    bgb = jnp.concatenate([b_gamma1, b_beta1, b_gamma2, b_beta2], axis=1)
    w2cat = jnp.concatenate([weight2, W_add2, W_rev2], axis=1)

    msg1p, h1p, gb2p, qap = _stage_a(x, d2, wall, pe, wgb, bgb)
    comb1 = _sc_agg_hist(msg1p.reshape(NP, 16), adj, idx)
    comb1p = comb1.reshape(5 * NP8, 128)
    msg2p, h2p, qb2p = _stage_c(h1p, comb1p, gb2p, qap, w2cat)
    comb2 = _sc_agg(msg2p.reshape(NP, 16), adj)
    logp, bacc, facc = _stage_d(h2p, comb2.reshape(2 * NP8, 128), comb1p,
                                qap, qb2p, W_fc, b_fc.reshape(1, C))
    return logp, bacc[0, 0], facc[0, 0]


# speedup vs baseline: 1.0651x; 1.0651x over previous
import functools

import numpy as np
import jax
import jax.numpy as jnp
from jax import lax
from jax.experimental import pallas as pl
from jax.experimental.pallas import tpu as pltpu
from jax.experimental.pallas import tpu_sc as plsc

N = 10000
E = 320000
F = 128
H1 = 16
H2 = 8
C = 8
DIMD = 64
DMAX = 1000
OMEGA = 0.1
K_THRESH = 32.0

NC = 2
NS = 16
NW = NC * NS
EPW = E // NW
CH = 80
NCHUNK = EPW // CH
NCHUNK_N = N // CH
CPT = -(-NCHUNK_N // NS)

BN = 1000
NBLK = N // BN
BN8 = BN // 8


def _make_pe(d_max, dim):
    pos = np.arange(d_max)[:, None].astype(np.float32)
    div = np.exp(np.arange(0, dim, 2).astype(np.float32) * -(np.log(10000.0) / dim))
    pe = np.zeros((d_max, dim), dtype=np.float32)
    pe[:, 0::2] = np.sin(pos * div)
    pe[:, 1::2] = np.cos(pos * div)
    return pe

_PE = _make_pe(DMAX, DIMD)

_F32 = jnp.float32


def _zero_shared(zbuf, acc, sid):
    z16 = jnp.zeros((16,), _F32)

    @pl.loop(0, CH)
    def _(i):
        zbuf[i] = z16

    @pl.loop(0, CPT)
    def _(k):
        g = sid + k * NS

        @pl.when(g < NCHUNK_N)
        def _():
            pltpu.sync_copy(zbuf, acc.at[pl.ds(g * CH, CH)])


def _writeback(acc, out, sid, sec):
    @pl.loop(0, CPT)
    def _(k):
        g = sid + k * NS

        @pl.when(g < NCHUNK_N)
        def _():
            pltpu.sync_copy(acc.at[pl.ds(g * CH, CH)],
                            out.at[pl.ds(sec * N + g * CH, CH)])


NBUF = 4
NQ = (NCHUNK - 1) // NBUF


def _sc_agg_hist_body(msg_hbm, adj_hbm, idx_hbm, out_hbm,
                      adjb, rows, idxb, ones_c, ones_i, zbuf,
                      acc, acc_cnt, accw, si, sg, ss, st):
    cid = lax.axis_index("c")
    sid = lax.axis_index("s")
    wid = cid * NS + sid
    base = wid * EPW

    e0 = jnp.where(lax.iota(jnp.int32, 16) == 0, 1.0, 0.0).astype(_F32)

    @pl.loop(0, CH)
    def _(i):
        ones_c[i] = e0

    _zero_shared(zbuf, acc, sid)
    _zero_shared(zbuf, acc_cnt, sid)

    @pl.when(cid == 0)
    def _():
        _zero_shared(zbuf, accw, sid)

    plsc.subcore_barrier()

    def idx_dma(c, s):
        return pltpu.make_async_copy(
            adj_hbm.at[:, pl.ds(base + c * CH, CH)], adjb.at[s], si.at[s])

    def gat_dma(s):
        return pltpu.make_async_copy(
            msg_hbm.at[adjb.at[s, 0]], rows.at[s], sg.at[s])

    def scat_dma(s):
        return pltpu.make_async_copy(
            rows.at[s], acc.at[adjb.at[s, 1]], ss.at[s])

    def cnt_dma(s):
        return pltpu.make_async_copy(
            ones_c, acc_cnt.at[adjb.at[s, 1]], st.at[s])

    for s in range(NBUF):
        idx_dma(s, s).start()

    @pl.loop(0, NQ)
    def _(q):
        c0 = q * NBUF
        for s in range(NBUF):
            idx_dma(c0 + s, s).wait()
            pltpu.async_copy(msg_hbm.at[adjb.at[s, 0]], rows.at[s], sg.at[s])
            pltpu.async_copy(ones_c, acc_cnt.at[adjb.at[s, 1]], st.at[s],
                             add=True)
        for s in range(NBUF):
            gat_dma(s).wait()
            pltpu.async_copy(rows.at[s], acc.at[adjb.at[s, 1]], ss.at[s],
                             add=True)
        for s in range(NBUF):
            scat_dma(s).wait()
            cnt_dma(s).wait()

            @pl.when(q < NQ - 1)
            def _():
                idx_dma(c0 + NBUF + s, s).start()

    idx_dma(NCHUNK - 1, 0).start()
    idx_dma(NCHUNK - 1, 0).wait()
    pltpu.async_copy(msg_hbm.at[adjb.at[0, 0]], rows.at[0], sg.at[0])
    pltpu.async_copy(ones_c, acc_cnt.at[adjb.at[0, 1]], st.at[0], add=True)
    gat_dma(0).wait()
    pltpu.async_copy(rows.at[0], acc.at[adjb.at[0, 1]], ss.at[0], add=True)
    scat_dma(0).wait()
    cnt_dma(0).wait()

    @pl.when(cid == 0)
    def _():
        @pl.loop(0, 40)
        def _(i):
            ones_i[i] = e0

        for c in (sid, sid + NS):
            @pl.when(c < 25)
            def _():
                pltpu.sync_copy(idx_hbm.at[pl.ds(c * 40, 40)], idxb)
                pltpu.sync_copy(ones_i, accw.at[idxb], add=True)

    plsc.subcore_barrier()

    @pl.when(cid == 0)
    def _():
        _writeback(acc, out_hbm, sid, 0)
        _writeback(acc_cnt, out_hbm, sid, 2)
        _writeback(accw, out_hbm, sid, 4)

    @pl.when(cid == 1)
    def _():
        _writeback(acc, out_hbm, sid, 1)
        _writeback(acc_cnt, out_hbm, sid, 3)


def _sc_agg_body(msg_hbm, adj_hbm, out_hbm,
                 adjb, rows, zbuf, acc, si, sg, ss):
    cid = lax.axis_index("c")
    sid = lax.axis_index("s")
    wid = cid * NS + sid
    base = wid * EPW

    _zero_shared(zbuf, acc, sid)
    plsc.subcore_barrier()

    def idx_dma(c, s):
        return pltpu.make_async_copy(
            adj_hbm.at[:, pl.ds(base + c * CH, CH)], adjb.at[s], si.at[s])

    def gat_dma(s):
        return pltpu.make_async_copy(
            msg_hbm.at[adjb.at[s, 0]], rows.at[s], sg.at[s])

    def scat_dma(s):
        return pltpu.make_async_copy(
            rows.at[s], acc.at[adjb.at[s, 1]], ss.at[s])

    for s in range(NBUF):
        idx_dma(s, s).start()

    @pl.loop(0, NQ)
    def _(q):
        c0 = q * NBUF
        for s in range(NBUF):
            idx_dma(c0 + s, s).wait()
            pltpu.async_copy(msg_hbm.at[adjb.at[s, 0]], rows.at[s], sg.at[s])
        for s in range(NBUF):
            gat_dma(s).wait()
            pltpu.async_copy(rows.at[s], acc.at[adjb.at[s, 1]], ss.at[s],
                             add=True)
        for s in range(NBUF):
            scat_dma(s).wait()

            @pl.when(q < NQ - 1)
            def _():
                idx_dma(c0 + NBUF + s, s).start()

    idx_dma(NCHUNK - 1, 0).start()
    idx_dma(NCHUNK - 1, 0).wait()
    pltpu.async_copy(msg_hbm.at[adjb.at[0, 0]], rows.at[0], sg.at[0])
    gat_dma(0).wait()
    pltpu.async_copy(rows.at[0], acc.at[adjb.at[0, 1]], ss.at[0], add=True)
    scat_dma(0).wait()

    plsc.subcore_barrier()

    @pl.when(cid == 0)
    def _():
        _writeback(acc, out_hbm, sid, 0)

    @pl.when(cid == 1)
    def _():
        _writeback(acc, out_hbm, sid, 1)


@functools.lru_cache(maxsize=None)
def _sc_kernels():
    mesh = plsc.VectorSubcoreMesh(core_axis_name="c", subcore_axis_name="s")
    cp = pltpu.CompilerParams(use_tc_tiling_on_sc=False)
    agg_hist = pl.kernel(
        _sc_agg_hist_body,
        out_type=jax.ShapeDtypeStruct((5 * N, 16), _F32),
        mesh=mesh,
        scratch_types=[pltpu.VMEM((NBUF, 2, CH), jnp.int32),
                       pltpu.VMEM((NBUF, CH, 16), _F32),
                       pltpu.VMEM((40,), jnp.int32),
                       pltpu.VMEM((CH, 16), _F32),
                       pltpu.VMEM((40, 16), _F32),
                       pltpu.VMEM((CH, 16), _F32),
                       pltpu.VMEM_SHARED((N, 16), _F32),
                       pltpu.VMEM_SHARED((N, 16), _F32),
                       pltpu.VMEM_SHARED((N, 16), _F32),
                       pltpu.SemaphoreType.DMA((NBUF,)),
                       pltpu.SemaphoreType.DMA((NBUF,)),
                       pltpu.SemaphoreType.DMA((NBUF,)),
                       pltpu.SemaphoreType.DMA((NBUF,))],
        compiler_params=cp,
    )
    agg = pl.kernel(
        _sc_agg_body,
        out_type=jax.ShapeDtypeStruct((2 * N, 16), _F32),
        mesh=mesh,
        scratch_types=[pltpu.VMEM((NBUF, 2, CH), jnp.int32),
                       pltpu.VMEM((NBUF, CH, 16), _F32),
                       pltpu.VMEM((CH, 16), _F32),
                       pltpu.VMEM_SHARED((N, 16), _F32),
                       pltpu.SemaphoreType.DMA((NBUF,)),
                       pltpu.SemaphoreType.DMA((NBUF,)),
                       pltpu.SemaphoreType.DMA((NBUF,))],
        compiler_params=cp,
    )
    return agg_hist, agg


def _sc_agg_hist(msg, adj, idx):
    return _sc_kernels()[0](msg, adj, idx)


def _sc_agg(msg, adj):
    return _sc_kernels()[1](msg, adj)


def _elu(v):
    return jnp.where(v > 0, v, jnp.exp(v) - 1.0)


def _stage_a_body(x_ref, d_ref, wall_ref, pe_ref, wgb_ref, bgb_ref,
                  msg1_ref, h1_ref, gb2_ref, qa_ref):
    xb = x_ref[...]
    hxx = lax.dot_general(xb, wall_ref[...], (((1,), (0,)), ((), ())),
                          preferred_element_type=_F32)
    h = hxx[:, :H1]
    xa = hxx[:, H1:2 * H1]
    xr = hxx[:, 2 * H1:3 * H1]

    t = lax.dot_general(pe_ref[...], wgb_ref[...], (((1,), (0,)), ((), ())),
                        preferred_element_type=_F32)
    t = jnp.maximum(t + bgb_ref[...], 0.0)

    db = d_ref[...]
    oh = (db == lax.broadcasted_iota(jnp.int32, (BN, DMAX), 1)).astype(_F32)
    gb = lax.dot_general(oh, t, (((1,), (0,)), ((), ())),
                         preferred_element_type=_F32)
    g1 = gb[:, :H1]
    b1 = gb[:, H1:2 * H1]
    g2 = gb[:, 2 * H1:2 * H1 + H2]
    b2 = gb[:, 2 * H1 + H2:2 * H1 + 2 * H2]

    r = (db < int(K_THRESH)).astype(_F32)
    badd = g1 * xa + b1
    brev = g1 * xr + b1
    ra = r * badd
    rr = (1.0 - r) * brev

    msg1_ref[...] = h + OMEGA * (ra - rr)
    h1_ref[...] = h
    gb2_ref[...] = jnp.concatenate([g2, b2], axis=1)

    qb1 = jnp.sum(ra * ra, axis=1, keepdims=True) + \
        jnp.sum(rr * rr, axis=1, keepdims=True)
    qf1 = jnp.sum(g1 * g1, axis=1, keepdims=True) + \
        jnp.sum(b1 * b1, axis=1, keepdims=True)
    qf2 = jnp.sum(g2 * g2, axis=1, keepdims=True) + \
        jnp.sum(b2 * b2, axis=1, keepdims=True)
    qa_ref[...] = jnp.concatenate([qb1, qf1, qf2, r], axis=1)


def _stage_c_body(h1_ref, p1a_ref, p1b_ref, cpa_ref, cpb_ref, gb2_ref, qa_ref,
                  w2_ref, msg2_ref, h2_ref, qb2_ref):
    cnt = cpa_ref[:, 0:1] + cpb_ref[:, 0:1]
    agg1 = (p1a_ref[...] + p1b_ref[...]) / jnp.maximum(cnt, 1.0)
    h1 = _elu(jnp.concatenate([h1_ref[...], agg1], axis=1))
    hxx = lax.dot_general(h1, w2_ref[...], (((1,), (0,)), ((), ())),
                          preferred_element_type=_F32)
    h = hxx[:, :H2]
    xa = hxx[:, H2:2 * H2]
    xr = hxx[:, 2 * H2:3 * H2]

    g2 = gb2_ref[:, :H2]
    b2 = gb2_ref[:, H2:2 * H2]
    r = qa_ref[:, 3:4]
    badd = g2 * xa + b2
    brev = g2 * xr + b2
    ra = r * badd
    rr = (1.0 - r) * brev

    msg2 = h + OMEGA * (ra - rr)
    msg2_ref[...] = jnp.concatenate(
        [msg2, jnp.zeros((BN, 16 - H2), _F32)], axis=1)
    h2_ref[...] = h
    qb2_ref[...] = jnp.sum(ra * ra, axis=1, keepdims=True) + \
        jnp.sum(rr * rr, axis=1, keepdims=True)


def _stage_d_body(h2_ref, p2a_ref, p2b_ref, cpa_ref, cpb_ref, w_ref,
                  qa_ref, qb2_ref, wfc_ref, bfc_ref,
                  logp_ref, bacc_ref, facc_ref):
    cnt = cpa_ref[:, 0:1] + cpb_ref[:, 0:1]
    agg2 = (p2a_ref[:, :H2] + p2b_ref[:, :H2]) / jnp.maximum(cnt, 1.0)
    h2 = _elu(jnp.concatenate([h2_ref[...], agg2], axis=1))
    logits = lax.dot_general(h2, wfc_ref[...], (((1,), (0,)), ((), ())),
                             preferred_element_type=_F32) + bfc_ref[...]
    m = jnp.max(logits, axis=1, keepdims=True)
    s = logits - m
    lse = jnp.log(jnp.sum(jnp.exp(s), axis=1, keepdims=True))
    logp_ref[...] = s - lse

    @pl.when(pl.program_id(0) == 0)
    def _():
        bacc_ref[...] = jnp.zeros((1, 1), _F32)
        facc_ref[...] = jnp.zeros((1, 1), _F32)

    wv = w_ref[:, 0:1]
    bpart = jnp.sum(wv * qa_ref[:, 0:1], keepdims=True) / (1000.0 * H1) + \
        jnp.sum(wv * qb2_ref[...], keepdims=True) / (1000.0 * H2)
    fpart = jnp.sum(wv * qa_ref[:, 1:2], keepdims=True) / (1000.0 * H1) + \
        jnp.sum(wv * qa_ref[:, 2:3], keepdims=True) / (1000.0 * H2)
    bacc_ref[...] += bpart
    facc_ref[...] += fpart


def _nblock(width):
    return pl.BlockSpec((BN, width), lambda i: (i, 0))


def _sec(sec):
    return pl.BlockSpec((BN, 16), lambda i, s=sec: (s * NBLK + i, 0))


def _full(shape):
    return pl.BlockSpec(shape, lambda i: tuple(0 for _ in shape))


def _stage_a(x, d2, wall, pe, wgb, bgb):
    return pl.pallas_call(
        _stage_a_body,
        grid=(NBLK,),
        in_specs=[_nblock(F), _nblock(1), _full((F, 3 * H1)),
                  _full((DMAX, DIMD)), _full((DIMD, 48)), _full((1, 48))],
        out_specs=[_nblock(16), _nblock(16), _nblock(16), _nblock(4)],
        out_shape=[jax.ShapeDtypeStruct((N, 16), _F32),
                   jax.ShapeDtypeStruct((N, 16), _F32),
                   jax.ShapeDtypeStruct((N, 16), _F32),
                   jax.ShapeDtypeStruct((N, 4), _F32)],
    )(x, d2, wall, pe, wgb, bgb)


def _stage_c(h1pre, comb1, gb2, qa, w2cat):
    return pl.pallas_call(
        _stage_c_body,
        grid=(NBLK,),
        in_specs=[_nblock(16), _sec(0), _sec(1), _sec(2),
                  _sec(3), _nblock(16), _nblock(4), _full((2 * H1, 3 * H2))],
        out_specs=[_nblock(16), _nblock(H2), _nblock(1)],
        out_shape=[jax.ShapeDtypeStruct((N, 16), _F32),
                   jax.ShapeDtypeStruct((N, H2), _F32),
                   jax.ShapeDtypeStruct((N, 1), _F32)],
    )(h1pre, comb1, comb1, comb1, comb1, gb2, qa, w2cat)


def _stage_d(h2pre, comb2, comb1, qa, qb2, wfc, bfc):
    return pl.pallas_call(
        _stage_d_body,
        grid=(NBLK,),
        in_specs=[_nblock(H2), _sec(0), _sec(1), _sec(2),
                  _sec(3), _sec(4), _nblock(4), _nblock(1),
                  _full((2 * H2, C)), _full((1, C))],
        out_specs=[_nblock(C),
                   pl.BlockSpec((1, 1), lambda i: (0, 0)),
                   pl.BlockSpec((1, 1), lambda i: (0, 0))],
        out_shape=[jax.ShapeDtypeStruct((N, C), _F32),
                   jax.ShapeDtypeStruct((1, 1), _F32),
                   jax.ShapeDtypeStruct((1, 1), _F32)],
    )(h2pre, comb2, comb2, comb1, comb1, comb1, qa, qb2, wfc, bfc)


def kernel(x, adj, d, idx, edge, weight1, W_gamma1, W_beta1, b_gamma1,
           b_beta1, W_add1, W_rev1, weight2, W_gamma2, W_beta2, b_gamma2,
           b_beta2, W_add2, W_rev2, W_fc, b_fc):
    d2 = d.reshape(N, 1)
    pe = jnp.asarray(_PE)
    wall = jnp.concatenate([weight1, W_add1, W_rev1], axis=1)
    wgb = jnp.concatenate([W_gamma1, W_beta1, W_gamma2, W_beta2], axis=1)
    bgb = jnp.concatenate([b_gamma1, b_beta1, b_gamma2, b_beta2], axis=1)
    w2cat = jnp.concatenate([weight2, W_add2, W_rev2], axis=1)

    msg1, h1pre, gb2, qa = _stage_a(x, d2, wall, pe, wgb, bgb)
    comb1 = _sc_agg_hist(msg1, adj, idx)
    msg2, h2pre, qb2 = _stage_c(h1pre, comb1, gb2, qa, w2cat)
    comb2 = _sc_agg(msg2, adj)
    logp, bacc, facc = _stage_d(h2pre, comb2, comb1, qa, qb2,
                                W_fc, b_fc.reshape(1, C))
    return logp, bacc[0, 0], facc[0, 0]


# speedup vs baseline: 1.1648x; 1.0936x over previous
import functools

import numpy as np
import jax
import jax.numpy as jnp
from jax import lax
from jax.experimental import pallas as pl
from jax.experimental.pallas import tpu as pltpu
from jax.experimental.pallas import tpu_sc as plsc

N = 10000
E = 320000
F = 128
H1 = 16
H2 = 8
C = 8
DIMD = 64
DMAX = 1000
OMEGA = 0.1
K_THRESH = 32.0

NC = 2
NS = 16
NW = NC * NS
EPW = E // NW
CH = 128
NFULL = EPW // CH
CT = EPW - NFULL * CH
ZCH = 80
NCHUNK_N = N // ZCH
CPT = -(-NCHUNK_N // NS)

BN = 1000
NBLK = N // BN
BN8 = BN // 8


def _make_pe(d_max, dim):
    pos = np.arange(d_max)[:, None].astype(np.float32)
    div = np.exp(np.arange(0, dim, 2).astype(np.float32) * -(np.log(10000.0) / dim))
    pe = np.zeros((d_max, dim), dtype=np.float32)
    pe[:, 0::2] = np.sin(pos * div)
    pe[:, 1::2] = np.cos(pos * div)
    return pe

_PE = _make_pe(DMAX, DIMD)

_F32 = jnp.float32


def _zero_shared(zbuf, acc, sid):
    z16 = jnp.zeros((16,), _F32)

    @pl.loop(0, ZCH)
    def _(i):
        zbuf[i] = z16

    @pl.loop(0, CPT)
    def _(k):
        g = sid + k * NS

        @pl.when(g < NCHUNK_N)
        def _():
            pltpu.sync_copy(zbuf, acc.at[pl.ds(g * ZCH, ZCH)])


def _writeback(acc, out, sid, sec):
    @pl.loop(0, CPT)
    def _(k):
        g = sid + k * NS

        @pl.when(g < NCHUNK_N)
        def _():
            pltpu.sync_copy(acc.at[pl.ds(g * ZCH, ZCH)],
                            out.at[pl.ds(sec * N + g * ZCH, ZCH)])


NBUF = 4
NQ = (NFULL - 2) // NBUF


def _sc_agg_hist_body(msg_hbm, adj_hbm, idx_hbm, out_hbm,
                      adjb, adjt, rows, rowst, idxb, ones_c, ones_i, zbuf,
                      acc, acc_cnt, accw, si, sg, ss, st):
    cid = lax.axis_index("c")
    sid = lax.axis_index("s")
    wid = cid * NS + sid
    base = wid * EPW

    e0 = jnp.where(lax.iota(jnp.int32, 16) == 0, 1.0, 0.0).astype(_F32)

    @pl.loop(0, CH)
    def _(i):
        ones_c[i] = e0

    _zero_shared(zbuf, acc, sid)
    _zero_shared(zbuf, acc_cnt, sid)

    @pl.when(cid == 0)
    def _():
        _zero_shared(zbuf, accw, sid)

    plsc.subcore_barrier()

    def idx_dma(c, s):
        return pltpu.make_async_copy(
            adj_hbm.at[:, pl.ds(base + c * CH, CH)], adjb.at[s], si.at[s])

    def gat_dma(s):
        return pltpu.make_async_copy(
            msg_hbm.at[adjb.at[s, 0]], rows.at[s], sg.at[s])

    def scat_dma(s):
        return pltpu.make_async_copy(
            rows.at[s], acc.at[adjb.at[s, 1]], ss.at[s])

    def cnt_dma(s):
        return pltpu.make_async_copy(
            ones_c, acc_cnt.at[adjb.at[s, 1]], st.at[s])

    for s in range(NBUF):
        idx_dma(s, s).start()

    @pl.loop(0, NQ)
    def _(q):
        c0 = q * NBUF
        for s in range(NBUF):
            idx_dma(c0 + s, s).wait()
            pltpu.async_copy(msg_hbm.at[adjb.at[s, 0]], rows.at[s], sg.at[s])
            pltpu.async_copy(ones_c, acc_cnt.at[adjb.at[s, 1]], st.at[s],
                             add=True)
        for s in range(NBUF):
            gat_dma(s).wait()
            pltpu.async_copy(rows.at[s], acc.at[adjb.at[s, 1]], ss.at[s],
                             add=True)
        for s in range(NBUF):
            scat_dma(s).wait()
            cnt_dma(s).wait()

            @pl.when(q < NQ - 1)
            def _():
                idx_dma(c0 + NBUF + s, s).start()

    for c, s in ((NQ * NBUF, 0), (NQ * NBUF + 1, 1)):
        idx_dma(c, s).start()
    for c, s in ((NQ * NBUF, 0), (NQ * NBUF + 1, 1)):
        idx_dma(c, s).wait()
        pltpu.async_copy(msg_hbm.at[adjb.at[s, 0]], rows.at[s], sg.at[s])
        pltpu.async_copy(ones_c, acc_cnt.at[adjb.at[s, 1]], st.at[s],
                         add=True)
    pltpu.sync_copy(adj_hbm.at[:, pl.ds(base + NFULL * CH, CT)], adjt)
    pltpu.async_copy(msg_hbm.at[adjt.at[0]], rowst, sg.at[2])
    pltpu.async_copy(ones_c.at[pl.ds(0, CT)], acc_cnt.at[adjt.at[1]],
                     st.at[2], add=True)
    for s in (0, 1):
        gat_dma(s).wait()
        pltpu.async_copy(rows.at[s], acc.at[adjb.at[s, 1]], ss.at[s],
                         add=True)
    pltpu.make_async_copy(msg_hbm.at[adjt.at[0]], rowst, sg.at[2]).wait()
    pltpu.async_copy(rowst, acc.at[adjt.at[1]], ss.at[2], add=True)
    for s in (0, 1):
        scat_dma(s).wait()
        cnt_dma(s).wait()
    pltpu.make_async_copy(rowst, acc.at[adjt.at[1]], ss.at[2]).wait()
    pltpu.make_async_copy(ones_c.at[pl.ds(0, CT)], acc_cnt.at[adjt.at[1]],
                          st.at[2]).wait()

    @pl.when(cid == 0)
    def _():
        @pl.loop(0, 40)
        def _(i):
            ones_i[i] = e0

        for c in (sid, sid + NS):
            @pl.when(c < 25)
            def _():
                pltpu.sync_copy(idx_hbm.at[pl.ds(c * 40, 40)], idxb)
                pltpu.sync_copy(ones_i, accw.at[idxb], add=True)

    plsc.subcore_barrier()

    @pl.when(cid == 0)
    def _():
        _writeback(acc, out_hbm, sid, 0)
        _writeback(acc_cnt, out_hbm, sid, 2)
        _writeback(accw, out_hbm, sid, 4)

    @pl.when(cid == 1)
    def _():
        _writeback(acc, out_hbm, sid, 1)
        _writeback(acc_cnt, out_hbm, sid, 3)


def _sc_agg_body(msg_hbm, adj_hbm, out_hbm,
                 adjb, adjt, rows, rowst, zbuf, acc, si, sg, ss):
    cid = lax.axis_index("c")
    sid = lax.axis_index("s")
    wid = cid * NS + sid
    base = wid * EPW

    _zero_shared(zbuf, acc, sid)
    plsc.subcore_barrier()

    def idx_dma(c, s):
        return pltpu.make_async_copy(
            adj_hbm.at[:, pl.ds(base + c * CH, CH)], adjb.at[s], si.at[s])

    def gat_dma(s):
        return pltpu.make_async_copy(
            msg_hbm.at[adjb.at[s, 0]], rows.at[s], sg.at[s])

    def scat_dma(s):
        return pltpu.make_async_copy(
            rows.at[s], acc.at[adjb.at[s, 1]], ss.at[s])

    for s in range(NBUF):
        idx_dma(s, s).start()

    @pl.loop(0, NQ)
    def _(q):
        c0 = q * NBUF
        for s in range(NBUF):
            idx_dma(c0 + s, s).wait()
            pltpu.async_copy(msg_hbm.at[adjb.at[s, 0]], rows.at[s], sg.at[s])
        for s in range(NBUF):
            gat_dma(s).wait()
            pltpu.async_copy(rows.at[s], acc.at[adjb.at[s, 1]], ss.at[s],
                             add=True)
        for s in range(NBUF):
            scat_dma(s).wait()

            @pl.when(q < NQ - 1)
            def _():
                idx_dma(c0 + NBUF + s, s).start()

    for c, s in ((NQ * NBUF, 0), (NQ * NBUF + 1, 1)):
        idx_dma(c, s).start()
    for c, s in ((NQ * NBUF, 0), (NQ * NBUF + 1, 1)):
        idx_dma(c, s).wait()
        pltpu.async_copy(msg_hbm.at[adjb.at[s, 0]], rows.at[s], sg.at[s])
    pltpu.sync_copy(adj_hbm.at[:, pl.ds(base + NFULL * CH, CT)], adjt)
    pltpu.async_copy(msg_hbm.at[adjt.at[0]], rowst, sg.at[2])
    for s in (0, 1):
        gat_dma(s).wait()
        pltpu.async_copy(rows.at[s], acc.at[adjb.at[s, 1]], ss.at[s],
                         add=True)
    pltpu.make_async_copy(msg_hbm.at[adjt.at[0]], rowst, sg.at[2]).wait()
    pltpu.async_copy(rowst, acc.at[adjt.at[1]], ss.at[2], add=True)
    for s in (0, 1):
        scat_dma(s).wait()
    pltpu.make_async_copy(rowst, acc.at[adjt.at[1]], ss.at[2]).wait()

    plsc.subcore_barrier()

    @pl.when(cid == 0)
    def _():
        _writeback(acc, out_hbm, sid, 0)

    @pl.when(cid == 1)
    def _():
        _writeback(acc, out_hbm, sid, 1)


@functools.lru_cache(maxsize=None)
def _sc_kernels():
    mesh = plsc.VectorSubcoreMesh(core_axis_name="c", subcore_axis_name="s")
    cp = pltpu.CompilerParams(use_tc_tiling_on_sc=False)
    agg_hist = pl.kernel(
        _sc_agg_hist_body,
        out_type=jax.ShapeDtypeStruct((5 * N, 16), _F32),
        mesh=mesh,
        scratch_types=[pltpu.VMEM((NBUF, 2, CH), jnp.int32),
                       pltpu.VMEM((2, CT), jnp.int32),
                       pltpu.VMEM((NBUF, CH, 16), _F32),
                       pltpu.VMEM((CT, 16), _F32),
                       pltpu.VMEM((40,), jnp.int32),
                       pltpu.VMEM((CH, 16), _F32),
                       pltpu.VMEM((40, 16), _F32),
                       pltpu.VMEM((ZCH, 16), _F32),
                       pltpu.VMEM_SHARED((N, 16), _F32),
                       pltpu.VMEM_SHARED((N, 16), _F32),
                       pltpu.VMEM_SHARED((N, 16), _F32),
                       pltpu.SemaphoreType.DMA((NBUF,)),
                       pltpu.SemaphoreType.DMA((NBUF,)),
                       pltpu.SemaphoreType.DMA((NBUF,)),
                       pltpu.SemaphoreType.DMA((NBUF,))],
        compiler_params=cp,
    )
    agg = pl.kernel(
        _sc_agg_body,
        out_type=jax.ShapeDtypeStruct((2 * N, 16), _F32),
        mesh=mesh,
        scratch_types=[pltpu.VMEM((NBUF, 2, CH), jnp.int32),
                       pltpu.VMEM((2, CT), jnp.int32),
                       pltpu.VMEM((NBUF, CH, 16), _F32),
                       pltpu.VMEM((CT, 16), _F32),
                       pltpu.VMEM((ZCH, 16), _F32),
                       pltpu.VMEM_SHARED((N, 16), _F32),
                       pltpu.SemaphoreType.DMA((NBUF,)),
                       pltpu.SemaphoreType.DMA((NBUF,)),
                       pltpu.SemaphoreType.DMA((NBUF,))],
        compiler_params=cp,
    )
    return agg_hist, agg


def _sc_agg_hist(msg, adj, idx):
    return _sc_kernels()[0](msg, adj, idx)


def _sc_agg(msg, adj):
    return _sc_kernels()[1](msg, adj)


def _elu(v):
    return jnp.where(v > 0, v, jnp.exp(v) - 1.0)


def _stage_a_body(x_ref, d_ref, wall_ref, pe_ref, wgb_ref, bgb_ref,
                  msg1_ref, h1_ref, gb2_ref, qa_ref):
    xb = x_ref[...]
    hxx = lax.dot_general(xb, wall_ref[...], (((1,), (0,)), ((), ())),
                          preferred_element_type=_F32)
    h = hxx[:, :H1]
    xa = hxx[:, H1:2 * H1]
    xr = hxx[:, 2 * H1:3 * H1]

    t = lax.dot_general(pe_ref[...], wgb_ref[...], (((1,), (0,)), ((), ())),
                        preferred_element_type=_F32)
    t = jnp.maximum(t + bgb_ref[...], 0.0)

    db = d_ref[...]
    oh = (db == lax.broadcasted_iota(jnp.int32, (BN, DMAX), 1)).astype(_F32)
    gb = lax.dot_general(oh, t, (((1,), (0,)), ((), ())),
                         preferred_element_type=_F32)
    g1 = gb[:, :H1]
    b1 = gb[:, H1:2 * H1]
    g2 = gb[:, 2 * H1:2 * H1 + H2]
    b2 = gb[:, 2 * H1 + H2:2 * H1 + 2 * H2]

    r = (db < int(K_THRESH)).astype(_F32)
    badd = g1 * xa + b1
    brev = g1 * xr + b1
    ra = r * badd
    rr = (1.0 - r) * brev

    msg1_ref[...] = h + OMEGA * (ra - rr)
    h1_ref[...] = h
    gb2_ref[...] = jnp.concatenate([g2, b2], axis=1)

    qb1 = jnp.sum(ra * ra, axis=1, keepdims=True) + \
        jnp.sum(rr * rr, axis=1, keepdims=True)
    qf1 = jnp.sum(g1 * g1, axis=1, keepdims=True) + \
        jnp.sum(b1 * b1, axis=1, keepdims=True)
    qf2 = jnp.sum(g2 * g2, axis=1, keepdims=True) + \
        jnp.sum(b2 * b2, axis=1, keepdims=True)
    qa_ref[...] = jnp.concatenate([qb1, qf1, qf2, r], axis=1)


def _stage_c_body(h1_ref, p1a_ref, p1b_ref, cpa_ref, cpb_ref, gb2_ref, qa_ref,
                  w2_ref, msg2_ref, h2_ref, qb2_ref):
    cnt = cpa_ref[:, 0:1] + cpb_ref[:, 0:1]
    agg1 = (p1a_ref[...] + p1b_ref[...]) / jnp.maximum(cnt, 1.0)
    h1 = _elu(jnp.concatenate([h1_ref[...], agg1], axis=1))
    hxx = lax.dot_general(h1, w2_ref[...], (((1,), (0,)), ((), ())),
                          preferred_element_type=_F32)
    h = hxx[:, :H2]
    xa = hxx[:, H2:2 * H2]
    xr = hxx[:, 2 * H2:3 * H2]

    g2 = gb2_ref[:, :H2]
    b2 = gb2_ref[:, H2:2 * H2]
    r = qa_ref[:, 3:4]
    badd = g2 * xa + b2
    brev = g2 * xr + b2
    ra = r * badd
    rr = (1.0 - r) * brev

    msg2 = h + OMEGA * (ra - rr)
    msg2_ref[...] = jnp.concatenate(
        [msg2, jnp.zeros((BN, 16 - H2), _F32)], axis=1)
    h2_ref[...] = h
    qb2_ref[...] = jnp.sum(ra * ra, axis=1, keepdims=True) + \
        jnp.sum(rr * rr, axis=1, keepdims=True)


def _stage_d_body(h2_ref, p2a_ref, p2b_ref, cpa_ref, cpb_ref, w_ref,
                  qa_ref, qb2_ref, wfc_ref, bfc_ref,
                  logp_ref, bacc_ref, facc_ref):
    cnt = cpa_ref[:, 0:1] + cpb_ref[:, 0:1]
    agg2 = (p2a_ref[:, :H2] + p2b_ref[:, :H2]) / jnp.maximum(cnt, 1.0)
    h2 = _elu(jnp.concatenate([h2_ref[...], agg2], axis=1))
    logits = lax.dot_general(h2, wfc_ref[...], (((1,), (0,)), ((), ())),
                             preferred_element_type=_F32) + bfc_ref[...]
    m = jnp.max(logits, axis=1, keepdims=True)
    s = logits - m
    lse = jnp.log(jnp.sum(jnp.exp(s), axis=1, keepdims=True))
    logp_ref[...] = s - lse

    @pl.when(pl.program_id(0) == 0)
    def _():
        bacc_ref[...] = jnp.zeros((1, 1), _F32)
        facc_ref[...] = jnp.zeros((1, 1), _F32)

    wv = w_ref[:, 0:1]
    bpart = jnp.sum(wv * qa_ref[:, 0:1], keepdims=True) / (1000.0 * H1) + \
        jnp.sum(wv * qb2_ref[...], keepdims=True) / (1000.0 * H2)
    fpart = jnp.sum(wv * qa_ref[:, 1:2], keepdims=True) / (1000.0 * H1) + \
        jnp.sum(wv * qa_ref[:, 2:3], keepdims=True) / (1000.0 * H2)
    bacc_ref[...] += bpart
    facc_ref[...] += fpart


def _nblock(width):
    return pl.BlockSpec((BN, width), lambda i: (i, 0))


def _sec(sec):
    return pl.BlockSpec((BN, 16), lambda i, s=sec: (s * NBLK + i, 0))


def _full(shape):
    return pl.BlockSpec(shape, lambda i: tuple(0 for _ in shape))


def _stage_a(x, d2, wall, pe, wgb, bgb):
    return pl.pallas_call(
        _stage_a_body,
        grid=(NBLK,),
        in_specs=[_nblock(F), _nblock(1), _full((F, 3 * H1)),
                  _full((DMAX, DIMD)), _full((DIMD, 48)), _full((1, 48))],
        out_specs=[_nblock(16), _nblock(16), _nblock(16), _nblock(4)],
        out_shape=[jax.ShapeDtypeStruct((N, 16), _F32),
                   jax.ShapeDtypeStruct((N, 16), _F32),
                   jax.ShapeDtypeStruct((N, 16), _F32),
                   jax.ShapeDtypeStruct((N, 4), _F32)],
    )(x, d2, wall, pe, wgb, bgb)


def _stage_c(h1pre, comb1, gb2, qa, w2cat):
    return pl.pallas_call(
        _stage_c_body,
        grid=(NBLK,),
        in_specs=[_nblock(16), _sec(0), _sec(1), _sec(2),
                  _sec(3), _nblock(16), _nblock(4), _full((2 * H1, 3 * H2))],
        out_specs=[_nblock(16), _nblock(H2), _nblock(1)],
        out_shape=[jax.ShapeDtypeStruct((N, 16), _F32),
                   jax.ShapeDtypeStruct((N, H2), _F32),
                   jax.ShapeDtypeStruct((N, 1), _F32)],
    )(h1pre, comb1, comb1, comb1, comb1, gb2, qa, w2cat)


def _stage_d(h2pre, comb2, comb1, qa, qb2, wfc, bfc):
    return pl.pallas_call(
        _stage_d_body,
        grid=(NBLK,),
        in_specs=[_nblock(H2), _sec(0), _sec(1), _sec(2),
                  _sec(3), _sec(4), _nblock(4), _nblock(1),
                  _full((2 * H2, C)), _full((1, C))],
        out_specs=[_nblock(C),
                   pl.BlockSpec((1, 1), lambda i: (0, 0)),
                   pl.BlockSpec((1, 1), lambda i: (0, 0))],
        out_shape=[jax.ShapeDtypeStruct((N, C), _F32),
                   jax.ShapeDtypeStruct((1, 1), _F32),
                   jax.ShapeDtypeStruct((1, 1), _F32)],
    )(h2pre, comb2, comb2, comb1, comb1, comb1, qa, qb2, wfc, bfc)


def kernel(x, adj, d, idx, edge, weight1, W_gamma1, W_beta1, b_gamma1,
           b_beta1, W_add1, W_rev1, weight2, W_gamma2, W_beta2, b_gamma2,
           b_beta2, W_add2, W_rev2, W_fc, b_fc):
    d2 = d.reshape(N, 1)
    pe = jnp.asarray(_PE)
    wall = jnp.concatenate([weight1, W_add1, W_rev1], axis=1)
    wgb = jnp.concatenate([W_gamma1, W_beta1, W_gamma2, W_beta2], axis=1)
    bgb = jnp.concatenate([b_gamma1, b_beta1, b_gamma2, b_beta2], axis=1)
    w2cat = jnp.concatenate([weight2, W_add2, W_rev2], axis=1)

    msg1, h1pre, gb2, qa = _stage_a(x, d2, wall, pe, wgb, bgb)
    comb1 = _sc_agg_hist(msg1, adj, idx)
    msg2, h2pre, qb2 = _stage_c(h1pre, comb1, gb2, qa, w2cat)
    comb2 = _sc_agg(msg2, adj)
    logp, bacc, facc = _stage_d(h2pre, comb2, comb1, qa, qb2,
                                W_fc, b_fc.reshape(1, C))
    return logp, bacc[0, 0], facc[0, 0]
